# baseline (speedup 1.0000x reference)
import jax
import jax.numpy as jnp
from jax import lax
from jax.experimental import pallas as pl
from jax.experimental.pallas import tpu as pltpu
from jax.experimental.pallas import tpu_sc as plsc

N = 10000
E = 160000
D = 256
DE = 16
G = 64

NPAD = 10240
EPAD = 163840
NC = 2
NS = 16
NW = NC * NS
NPS = NPAD // NS
ECHUNK = EPAD // NW
EB = 16384
XB = 1024

_HIGH = lax.Precision.HIGHEST


def _xl1_body(x_ref, w_ref, o_ref):
    i = pl.program_id(0)
    n_glob = i * XB + lax.broadcasted_iota(jnp.int32, (1, XB), 1)
    o = lax.dot_general(w_ref[...], x_ref[...],
                        (((0,), (1,)), ((), ())),
                        precision=_HIGH)
    o_ref[...] = jnp.where(n_glob < N, o, 0.0)


EB_LAST = E - (EPAD // EB - 1) * EB


def _edge_w_body(ea_hbm, wcat, bcat, w2cat, b2cat, w1_ref, w2_ref,
                 ea_v, sem):
    i = pl.program_id(0)
    nb = EPAD // EB

    @pl.when(i < nb - 1)
    def _():
        pltpu.async_copy(ea_hbm.at[pl.ds(i * EB, EB)], ea_v, sem).wait()

    @pl.when(i == nb - 1)
    def _():
        src = ea_hbm.at[pl.ds((nb - 1) * EB, EB_LAST)]
        pltpu.async_copy(src, ea_v.at[pl.ds(0, EB_LAST)], sem).wait()

    e_glob = i * EB + lax.broadcasted_iota(jnp.int32, (1, EB), 1)
    valid = e_glob < E
    h = lax.dot_general(wcat[...], ea_v[...], (((0,), (1,)), ((), ())))
    h = jnp.maximum(h + bcat[...], 0.0)
    o = lax.dot_general(w2cat[...], h, (((0,), (0,)), ((), ())))
    o = jax.nn.sigmoid(o + b2cat[...])
    w1_ref[...] = jnp.where(valid, o[0:1, :], 0.0)
    w2_ref[...] = jnp.where(valid, o[1:2, :], 0.0)


def _mid_body(msg_ref, deg_ref, xlt_ref, batch_ref, b1_ref, gnw_ref, gnb_ref,
              gnms_ref, w2_ref, out_ref):
    h = (msg_ref[0] + msg_ref[1]
         + xlt_ref[...] / deg_ref[...]
         + b1_ref[...])
    ohT = (lax.broadcasted_iota(jnp.int32, (G, NPAD), 0)
           == batch_ref[...]).astype(jnp.float32)
    cnt = jnp.maximum(jnp.sum(ohT, axis=1), 1.0)[None, :]
    seg = lax.dot_general(h, ohT, (((1,), (1,)), ((), ())),
                          precision=_HIGH)
    mean = seg / cnt
    mean_b = lax.dot_general(mean, ohT, (((1,), (0,)), ((), ())),
                             precision=_HIGH)
    out = h - mean_b * gnms_ref[...]
    var = lax.dot_general(out * out, ohT, (((1,), (1,)), ((), ())),
                          precision=_HIGH) / cnt
    std = jnp.sqrt(var + 1e-5)
    std_b = lax.dot_general(std, ohT, (((1,), (0,)), ((), ())),
                            precision=_HIGH)
    std_b = jnp.where(std_b > 0.0, std_b, 1.0)
    hn = gnw_ref[...] * out / std_b + gnb_ref[...]
    hr = jnp.maximum(hn, 0.0)
    out_ref[...] = lax.dot_general(w2_ref[...], hr, (((0,), (0,)), ((), ())),
                                   precision=_HIGH)


def _final_body(msg_ref, deg_ref, xl2_ref, b2_ref, out_ref):
    h = (msg_ref[pl.ds(0, 1), :] + msg_ref[pl.ds(1, 1), :]
         + xl2_ref[...] / deg_ref[...] + b2_ref[...])
    out_ref[...] = jax.nn.sigmoid(h)


def _rsqrt16(x):
    i = plsc.bitcast(x, jnp.int32)
    i = jnp.int32(0x5F3759DF) - lax.shift_right_logical(i, 1)
    y = plsc.bitcast(i, jnp.float32)
    hx = 0.5 * x
    for _ in range(4):
        y = y * (1.5 - (hx * y) * y)
    return y


def _sc_msg1_body(row_h, col_h, w1_h, w2_h, xlt_h,
                  msg_o, deg1_o, deg2_o, wn2_o,
                  rowb, colb, w1b, w2b, wn1b, wn2b, valb,
                  dis1l, dis2l, xll, nodeb,
                  deg1s, deg2s, dis1s, dis2s, m0s, m1s, m2s, m3s, m4s):
    c = lax.axis_index("c")
    s = lax.axis_index("s")
    wid = c * NS + s
    nbase = s * NPS
    msgs = (m0s, m1s, m2s, m3s, m4s)

    def zero_loop(i, _):
        nodeb[pl.ds(i * 16, 16)] = jnp.zeros((16,), jnp.float32)
        return 0
    lax.fori_loop(0, NPS // 16, zero_loop, 0)
    for m in msgs:
        pltpu.sync_copy(nodeb, m.at[pl.ds(nbase, NPS)])

    def one_loop(i, _):
        nodeb[pl.ds(i * 16, 16)] = jnp.ones((16,), jnp.float32)
        return 0
    lax.fori_loop(0, NPS // 16, one_loop, 0)
    pltpu.sync_copy(nodeb, deg1s.at[pl.ds(nbase, NPS)])
    pltpu.sync_copy(nodeb, deg2s.at[pl.ds(nbase, NPS)])
    plsc.subcore_barrier()

    for h in range(2):
        dbase = s * (2 * ECHUNK) + h * ECHUNK
        pltpu.sync_copy(col_h.at[pl.ds(dbase, ECHUNK)], colb)
        pltpu.sync_copy(w1_h.at[pl.ds(dbase, ECHUNK)], w1b)
        pltpu.sync_copy(w1b, deg1s.at[colb], add=True)
        pltpu.sync_copy(w2_h.at[pl.ds(dbase, ECHUNK)], w2b)
        pltpu.sync_copy(w2b, deg2s.at[colb], add=True)
    plsc.subcore_barrier()

    @pl.when(c == 0)
    def _():
        pltpu.sync_copy(deg1s.at[pl.ds(nbase, NPS)],
                        deg1_o.at[pl.ds(nbase, NPS)])
        pltpu.sync_copy(deg2s.at[pl.ds(nbase, NPS)],
                        deg2_o.at[pl.ds(nbase, NPS)])

    for deg_s, dis_s in ((deg1s, dis1s), (deg2s, dis2s)):
        pltpu.sync_copy(deg_s.at[pl.ds(nbase, NPS)], nodeb)

        def rs_loop(i, _):
            sl = pl.ds(i * 16, 16)
            nodeb[sl] = _rsqrt16(nodeb[sl])
            return 0
        lax.fori_loop(0, NPS // 16, rs_loop, 0)
        pltpu.sync_copy(nodeb, dis_s.at[pl.ds(nbase, NPS)])
    plsc.subcore_barrier()

    pltpu.sync_copy(dis1s, dis1l)
    pltpu.sync_copy(dis2s, dis2l)
    pltpu.sync_copy(xlt_h, xll)

    ebase = wid * ECHUNK
    pltpu.sync_copy(row_h.at[pl.ds(ebase, ECHUNK)], rowb)
    pltpu.sync_copy(col_h.at[pl.ds(ebase, ECHUNK)], colb)
    pltpu.sync_copy(w1_h.at[pl.ds(ebase, ECHUNK)], w1b)
    pltpu.sync_copy(w2_h.at[pl.ds(ebase, ECHUNK)], w2b)

    def wn_loop(i, _):
        sl = pl.ds(i * 16, 16)
        r = rowb[sl]
        cc = colb[sl]
        wn1b[sl] = (w1b[sl] * plsc.load_gather(dis1l, [r])) \
            * plsc.load_gather(dis1l, [cc])
        wn2b[sl] = (w2b[sl] * plsc.load_gather(dis2l, [r])) \
            * plsc.load_gather(dis2l, [cc])
        return 0
    lax.fori_loop(0, ECHUNK // 16, wn_loop, 0)
    pltpu.sync_copy(wn2b, wn2_o.at[pl.ds(ebase, ECHUNK)])

    for f in range(5):
        foff = jnp.int32(f * NPAD)

        def msg_loop(i, _):
            sl = pl.ds(i * 16, 16)
            valb[sl] = wn1b[sl] * plsc.load_gather(xll, [rowb[sl] + foff])
            return 0
        lax.fori_loop(0, ECHUNK // 16, msg_loop, 0)
        pltpu.sync_copy(valb, msgs[f].at[colb], add=True)
    plsc.subcore_barrier()

    for f in range(5):
        moff = (c * 5 + f) * NPAD + nbase
        pltpu.sync_copy(msgs[f].at[pl.ds(nbase, NPS)],
                        msg_o.at[pl.ds(moff, NPS)])


def _sc_msg2_body(row_h, col_h, wn_h, xl2_h,
                  msg_o,
                  rowb, colb, wnb, valb, xl2l, nodeb,
                  m0s):
    c = lax.axis_index("c")
    s = lax.axis_index("s")
    wid = c * NS + s
    nbase = s * NPS

    def zero_loop(i, _):
        nodeb[pl.ds(i * 16, 16)] = jnp.zeros((16,), jnp.float32)
        return 0
    lax.fori_loop(0, NPS // 16, zero_loop, 0)
    pltpu.sync_copy(nodeb, m0s.at[pl.ds(nbase, NPS)])
    plsc.subcore_barrier()

    pltpu.sync_copy(xl2_h, xl2l)
    ebase = wid * ECHUNK
    pltpu.sync_copy(row_h.at[pl.ds(ebase, ECHUNK)], rowb)
    pltpu.sync_copy(col_h.at[pl.ds(ebase, ECHUNK)], colb)
    pltpu.sync_copy(wn_h.at[pl.ds(ebase, ECHUNK)], wnb)

    def msg_loop(i, _):
        sl = pl.ds(i * 16, 16)
        valb[sl] = wnb[sl] * plsc.load_gather(xl2l, [rowb[sl]])
        return 0
    lax.fori_loop(0, ECHUNK // 16, msg_loop, 0)
    pltpu.sync_copy(valb, m0s.at[colb], add=True)
    plsc.subcore_barrier()

    pltpu.sync_copy(m0s.at[pl.ds(nbase, NPS)],
                    msg_o.at[pl.ds(c * NPAD + nbase, NPS)])


def _sc_mesh():
    return plsc.VectorSubcoreMesh(core_axis_name="c", subcore_axis_name="s",
                                  num_cores=NC, num_subcores=NS)


_full_spec = lambda shp: pl.BlockSpec(shp, lambda: tuple(0 for _ in shp))


@jax.jit
def kernel(x, edge_index, edge_attr, batch_idx, W1, b1, mlp1_w1, mlp1_b1,
           mlp1_w2, mlp1_b2, gn_w, gn_b, gn_ms, W2, b2, mlp2_w1, mlp2_b1,
           mlp2_w2, mlp2_b2):
    f32 = jnp.float32

    row_p = jnp.pad(edge_index[0], (0, EPAD - E), constant_values=NPAD - 1)
    col_p = jnp.pad(edge_index[1], (0, EPAD - E), constant_values=NPAD - 1)
    batch_p = jnp.pad(batch_idx, (0, NPAD - N), constant_values=-1)[None, :]

    xlt = pl.pallas_call(
        _xl1_body,
        grid=(NPAD // XB,),
        in_specs=[pl.BlockSpec((XB, D), lambda i: (i, 0)),
                  pl.BlockSpec((D, 5), lambda i: (0, 0))],
        out_specs=pl.BlockSpec((5, XB), lambda i: (0, i)),
        out_shape=jax.ShapeDtypeStruct((5, NPAD), f32),
    )(x, W1)

    wcat = jnp.concatenate([mlp1_w1, mlp2_w1], axis=1)
    bcat = jnp.concatenate([mlp1_b1, mlp2_b1])[:, None]
    w2cat = jnp.zeros((10, 2), f32)
    w2cat = w2cat.at[:5, 0].set(mlp1_w2[:, 0]).at[5:, 1].set(mlp2_w2[:, 0])
    b2cat = jnp.concatenate([mlp1_b2, mlp2_b2])[:, None]

    wspec = pl.BlockSpec((1, EB), lambda i: (0, i))
    full = lambda shp: pl.BlockSpec(shp, lambda i: tuple(0 for _ in shp))
    w1e, w2e = pl.pallas_call(
        _edge_w_body,
        grid=(EPAD // EB,),
        in_specs=[pl.BlockSpec(memory_space=pl.ANY),
                  full((DE, 10)), full((10, 1)), full((10, 2)), full((2, 1))],
        out_specs=[wspec, wspec],
        out_shape=[jax.ShapeDtypeStruct((1, EPAD), f32),
                   jax.ShapeDtypeStruct((1, EPAD), f32)],
        scratch_shapes=[pltpu.VMEM((EB, DE), f32),
                        pltpu.SemaphoreType.DMA],
    )(edge_attr, wcat, bcat, w2cat, b2cat)
    w1e = w1e.reshape(EPAD)
    w2e = w2e.reshape(EPAD)

    sc1 = pl.kernel(
        _sc_msg1_body,
        out_type=[jax.ShapeDtypeStruct((NC * 5 * NPAD,), f32),
                  jax.ShapeDtypeStruct((NPAD,), f32),
                  jax.ShapeDtypeStruct((NPAD,), f32),
                  jax.ShapeDtypeStruct((EPAD,), f32)],
        mesh=_sc_mesh(),
        compiler_params=pltpu.CompilerParams(needs_layout_passes=False),
        scratch_types=[
            pltpu.VMEM((ECHUNK,), jnp.int32),
            pltpu.VMEM((ECHUNK,), jnp.int32),
            pltpu.VMEM((ECHUNK,), f32),
            pltpu.VMEM((ECHUNK,), f32),
            pltpu.VMEM((ECHUNK,), f32),
            pltpu.VMEM((ECHUNK,), f32),
            pltpu.VMEM((ECHUNK,), f32),
            pltpu.VMEM((NPAD,), f32),
            pltpu.VMEM((NPAD,), f32),
            pltpu.VMEM((5 * NPAD,), f32),
            pltpu.VMEM((NPS,), f32),
            pltpu.VMEM_SHARED((NPAD,), f32),
            pltpu.VMEM_SHARED((NPAD,), f32),
            pltpu.VMEM_SHARED((NPAD,), f32),
            pltpu.VMEM_SHARED((NPAD,), f32),
            pltpu.VMEM_SHARED((NPAD,), f32),
            pltpu.VMEM_SHARED((NPAD,), f32),
            pltpu.VMEM_SHARED((NPAD,), f32),
            pltpu.VMEM_SHARED((NPAD,), f32),
            pltpu.VMEM_SHARED((NPAD,), f32),
        ],
    )
    msg1, deg1, deg2, wn2 = sc1(row_p, col_p, w1e, w2e, xlt.reshape(5 * NPAD))
    msg1 = msg1.reshape(NC, 5, NPAD)
    deg1 = deg1[None, :]
    deg2 = deg2[None, :]

    xl2 = pl.pallas_call(
        _mid_body,
        in_specs=[
            _full_spec((NC, 5, NPAD)),
            _full_spec((1, NPAD)),
            _full_spec((5, NPAD)),
            _full_spec((1, NPAD)),
            _full_spec((5, 1)),
            _full_spec((5, 1)),
            _full_spec((5, 1)),
            _full_spec((5, 1)),
            _full_spec((5, 1)),
        ],
        out_specs=_full_spec((1, NPAD)),
        out_shape=jax.ShapeDtypeStruct((1, NPAD), f32),
    )(msg1, deg1, xlt, batch_p, b1[:, None], gn_w[:, None],
      gn_b[:, None], gn_ms[:, None], W2)

    sc2 = pl.kernel(
        _sc_msg2_body,
        out_type=[jax.ShapeDtypeStruct((NC * NPAD,), f32)],
        mesh=_sc_mesh(),
        compiler_params=pltpu.CompilerParams(needs_layout_passes=False),
        scratch_types=[
            pltpu.VMEM((ECHUNK,), jnp.int32),
            pltpu.VMEM((ECHUNK,), jnp.int32),
            pltpu.VMEM((ECHUNK,), f32),
            pltpu.VMEM((ECHUNK,), f32),
            pltpu.VMEM((NPAD,), f32),
            pltpu.VMEM((NPS,), f32),
            pltpu.VMEM_SHARED((NPAD,), f32),
        ],
    )
    (msg2,) = sc2(row_p, col_p, wn2, xl2.reshape(NPAD))
    msg2 = msg2.reshape(NC, NPAD)

    out = pl.pallas_call(
        _final_body,
        in_specs=[_full_spec((NC, NPAD)), _full_spec((1, NPAD)),
                  _full_spec((1, NPAD)), _full_spec((1, 1))],
        out_specs=_full_spec((1, NPAD)),
        out_shape=jax.ShapeDtypeStruct((1, NPAD), f32),
    )(msg2, deg2, xl2, b2[:, None])

    return out[0, :N, None]


# speedup vs baseline: 1.0493x; 1.0493x over previous
import jax
import jax.numpy as jnp
from jax import lax
from jax.experimental import pallas as pl
from jax.experimental.pallas import tpu as pltpu
from jax.experimental.pallas import tpu_sc as plsc

N = 10000
E = 160000
D = 256
DE = 16
G = 64

NPAD = 10240
EPAD = 163840
NC = 2
NS = 16
NW = NC * NS
NPS = NPAD // NS
ECHUNK = EPAD // NW
EB = 16384
XB = 1024

_HIGH = lax.Precision.HIGHEST


XLAST = N - (NPAD // XB - 1) * XB


def _xl1_body(x_hbm, w_ref, o_ref, x_v, sem):
    i = pl.program_id(0)
    nb = NPAD // XB

    @pl.when(i < nb - 1)
    def _():
        pltpu.async_copy(x_hbm.at[pl.ds(i * XB, XB)], x_v, sem).wait()

    @pl.when(i == nb - 1)
    def _():
        src = x_hbm.at[pl.ds((nb - 1) * XB, XLAST)]
        pltpu.async_copy(src, x_v.at[pl.ds(0, XLAST)], sem).wait()

    n_glob = i * XB + lax.broadcasted_iota(jnp.int32, (1, XB), 1)
    o = lax.dot_general(w_ref[...], x_v[...],
                        (((0,), (1,)), ((), ())),
                        precision=_HIGH)
    o_ref[...] = jnp.where(n_glob < N, o, 0.0)


EB_LAST = E - (EPAD // EB - 1) * EB


def _edge_w_body(ea_ref, wcat, bcat, w2cat, b2cat, w1_ref, w2_ref):
    i = pl.program_id(0)
    e_glob = i * EB + lax.broadcasted_iota(jnp.int32, (1, EB), 1)
    valid = e_glob < E
    h = lax.dot_general(wcat[...], ea_ref[...], (((0,), (1,)), ((), ())))
    h = jnp.maximum(h + bcat[...], 0.0)
    o = lax.dot_general(w2cat[...], h, (((0,), (0,)), ((), ())))
    o = jax.nn.sigmoid(o + b2cat[...])
    w1_ref[...] = jnp.where(valid, o[0:1, :], 0.0)
    w2_ref[...] = jnp.where(valid, o[1:2, :], 0.0)


def _mid_body(msg_ref, deg_ref, xlt_ref, batch_ref, b1_ref, gnw_ref, gnb_ref,
              gnms_ref, w2_ref, out_ref):
    h = (msg_ref[0] + msg_ref[1]
         + xlt_ref[...] / deg_ref[...]
         + b1_ref[...])
    ohT = (lax.broadcasted_iota(jnp.int32, (G, NPAD), 0)
           == batch_ref[...]).astype(jnp.float32)
    cnt = jnp.maximum(jnp.sum(ohT, axis=1), 1.0)[None, :]
    seg = lax.dot_general(h, ohT, (((1,), (1,)), ((), ())),
                          precision=_HIGH)
    mean = seg / cnt
    mean_b = lax.dot_general(mean, ohT, (((1,), (0,)), ((), ())),
                             precision=_HIGH)
    out = h - mean_b * gnms_ref[...]
    var = lax.dot_general(out * out, ohT, (((1,), (1,)), ((), ())),
                          precision=_HIGH) / cnt
    std = jnp.sqrt(var + 1e-5)
    std_b = lax.dot_general(std, ohT, (((1,), (0,)), ((), ())),
                            precision=_HIGH)
    std_b = jnp.where(std_b > 0.0, std_b, 1.0)
    hn = gnw_ref[...] * out / std_b + gnb_ref[...]
    hr = jnp.maximum(hn, 0.0)
    out_ref[...] = lax.dot_general(w2_ref[...], hr, (((0,), (0,)), ((), ())),
                                   precision=_HIGH)


def _final_body(msg_ref, deg_ref, xl2_ref, b2_ref, out_ref):
    h = (msg_ref[pl.ds(0, 1), :] + msg_ref[pl.ds(1, 1), :]
         + xl2_ref[...] / deg_ref[...] + b2_ref[...])
    out_ref[...] = jax.nn.sigmoid(h)


def _rsqrt16(x):
    i = plsc.bitcast(x, jnp.int32)
    i = jnp.int32(0x5F3759DF) - lax.shift_right_logical(i, 1)
    y = plsc.bitcast(i, jnp.float32)
    hx = 0.5 * x
    for _ in range(4):
        y = y * (1.5 - (hx * y) * y)
    return y


def _sc_msg1_body(row_h, col_h, w1_h, w2_h, xlt_h,
                  msg_o, deg1_o, deg2_o, wn2_o,
                  rowb, colb, w1b, w2b, wn1b, wn2b, valb,
                  dis1l, dis2l, xll, nodeb,
                  deg1s, deg2s, dis1s, dis2s, m0s, m1s, m2s, m3s, m4s):
    c = lax.axis_index("c")
    s = lax.axis_index("s")
    wid = c * NS + s
    nbase = s * NPS
    msgs = (m0s, m1s, m2s, m3s, m4s)

    def zero_loop(i, _):
        nodeb[pl.ds(i * 16, 16)] = jnp.zeros((16,), jnp.float32)
        return 0
    lax.fori_loop(0, NPS // 16, zero_loop, 0)
    for m in msgs:
        pltpu.sync_copy(nodeb, m.at[pl.ds(nbase, NPS)])

    def one_loop(i, _):
        nodeb[pl.ds(i * 16, 16)] = jnp.ones((16,), jnp.float32)
        return 0
    lax.fori_loop(0, NPS // 16, one_loop, 0)
    pltpu.sync_copy(nodeb, deg1s.at[pl.ds(nbase, NPS)])
    pltpu.sync_copy(nodeb, deg2s.at[pl.ds(nbase, NPS)])
    plsc.subcore_barrier()

    for h in range(2):
        dbase = s * (2 * ECHUNK) + h * ECHUNK
        pltpu.sync_copy(col_h.at[pl.ds(dbase, ECHUNK)], colb)
        pltpu.sync_copy(w1_h.at[pl.ds(dbase, ECHUNK)], w1b)
        pltpu.sync_copy(w1b, deg1s.at[colb], add=True)
        pltpu.sync_copy(w2_h.at[pl.ds(dbase, ECHUNK)], w2b)
        pltpu.sync_copy(w2b, deg2s.at[colb], add=True)
    plsc.subcore_barrier()

    @pl.when(c == 0)
    def _():
        pltpu.sync_copy(deg1s.at[pl.ds(nbase, NPS)],
                        deg1_o.at[pl.ds(nbase, NPS)])
        pltpu.sync_copy(deg2s.at[pl.ds(nbase, NPS)],
                        deg2_o.at[pl.ds(nbase, NPS)])

    for deg_s, dis_s in ((deg1s, dis1s), (deg2s, dis2s)):
        pltpu.sync_copy(deg_s.at[pl.ds(nbase, NPS)], nodeb)

        def rs_loop(i, _):
            sl = pl.ds(i * 16, 16)
            nodeb[sl] = _rsqrt16(nodeb[sl])
            return 0
        lax.fori_loop(0, NPS // 16, rs_loop, 0)
        pltpu.sync_copy(nodeb, dis_s.at[pl.ds(nbase, NPS)])
    plsc.subcore_barrier()

    pltpu.sync_copy(dis1s, dis1l)
    pltpu.sync_copy(dis2s, dis2l)
    pltpu.sync_copy(xlt_h, xll)

    ebase = wid * ECHUNK
    pltpu.sync_copy(row_h.at[pl.ds(ebase, ECHUNK)], rowb)
    pltpu.sync_copy(col_h.at[pl.ds(ebase, ECHUNK)], colb)
    pltpu.sync_copy(w1_h.at[pl.ds(ebase, ECHUNK)], w1b)
    pltpu.sync_copy(w2_h.at[pl.ds(ebase, ECHUNK)], w2b)

    def wn_loop(i, _):
        sl = pl.ds(i * 16, 16)
        r = rowb[sl]
        cc = colb[sl]
        wn1b[sl] = (w1b[sl] * plsc.load_gather(dis1l, [r])) \
            * plsc.load_gather(dis1l, [cc])
        wn2b[sl] = (w2b[sl] * plsc.load_gather(dis2l, [r])) \
            * plsc.load_gather(dis2l, [cc])
        return 0
    lax.fori_loop(0, ECHUNK // 16, wn_loop, 0)
    pltpu.sync_copy(wn2b, wn2_o.at[pl.ds(ebase, ECHUNK)])

    for f in range(5):
        foff = jnp.int32(f * NPAD)

        def msg_loop(i, _):
            sl = pl.ds(i * 16, 16)
            valb[sl] = wn1b[sl] * plsc.load_gather(xll, [rowb[sl] + foff])
            return 0
        lax.fori_loop(0, ECHUNK // 16, msg_loop, 0)
        pltpu.sync_copy(valb, msgs[f].at[colb], add=True)
    plsc.subcore_barrier()

    for f in range(5):
        moff = (c * 5 + f) * NPAD + nbase
        pltpu.sync_copy(msgs[f].at[pl.ds(nbase, NPS)],
                        msg_o.at[pl.ds(moff, NPS)])


def _sc_msg2_body(row_h, col_h, wn_h, xl2_h,
                  msg_o,
                  rowb, colb, wnb, valb, xl2l, nodeb,
                  m0s):
    c = lax.axis_index("c")
    s = lax.axis_index("s")
    wid = c * NS + s
    nbase = s * NPS

    def zero_loop(i, _):
        nodeb[pl.ds(i * 16, 16)] = jnp.zeros((16,), jnp.float32)
        return 0
    lax.fori_loop(0, NPS // 16, zero_loop, 0)
    pltpu.sync_copy(nodeb, m0s.at[pl.ds(nbase, NPS)])
    plsc.subcore_barrier()

    pltpu.sync_copy(xl2_h, xl2l)
    ebase = wid * ECHUNK
    pltpu.sync_copy(row_h.at[pl.ds(ebase, ECHUNK)], rowb)
    pltpu.sync_copy(col_h.at[pl.ds(ebase, ECHUNK)], colb)
    pltpu.sync_copy(wn_h.at[pl.ds(ebase, ECHUNK)], wnb)

    def msg_loop(i, _):
        sl = pl.ds(i * 16, 16)
        valb[sl] = wnb[sl] * plsc.load_gather(xl2l, [rowb[sl]])
        return 0
    lax.fori_loop(0, ECHUNK // 16, msg_loop, 0)
    pltpu.sync_copy(valb, m0s.at[colb], add=True)
    plsc.subcore_barrier()

    pltpu.sync_copy(m0s.at[pl.ds(nbase, NPS)],
                    msg_o.at[pl.ds(c * NPAD + nbase, NPS)])


def _sc_mesh():
    return plsc.VectorSubcoreMesh(core_axis_name="c", subcore_axis_name="s",
                                  num_cores=NC, num_subcores=NS)


_full_spec = lambda shp: pl.BlockSpec(shp, lambda: tuple(0 for _ in shp))


@jax.jit
def kernel(x, edge_index, edge_attr, batch_idx, W1, b1, mlp1_w1, mlp1_b1,
           mlp1_w2, mlp1_b2, gn_w, gn_b, gn_ms, W2, b2, mlp2_w1, mlp2_b1,
           mlp2_w2, mlp2_b2):
    f32 = jnp.float32

    row_p = jnp.pad(edge_index[0], (0, EPAD - E), constant_values=NPAD - 1)
    col_p = jnp.pad(edge_index[1], (0, EPAD - E), constant_values=NPAD - 1)
    batch_p = jnp.pad(batch_idx, (0, NPAD - N), constant_values=-1)[None, :]

    xlt = pl.pallas_call(
        _xl1_body,
        grid=(NPAD // XB,),
        in_specs=[pl.BlockSpec(memory_space=pl.ANY),
                  pl.BlockSpec((D, 5), lambda i: (0, 0))],
        out_specs=pl.BlockSpec((5, XB), lambda i: (0, i)),
        out_shape=jax.ShapeDtypeStruct((5, NPAD), f32),
        scratch_shapes=[pltpu.VMEM((XB, D), f32),
                        pltpu.SemaphoreType.DMA],
    )(x, W1)

    wcat = jnp.concatenate([mlp1_w1, mlp2_w1], axis=1)
    bcat = jnp.concatenate([mlp1_b1, mlp2_b1])[:, None]
    w2cat = jnp.zeros((10, 2), f32)
    w2cat = w2cat.at[:5, 0].set(mlp1_w2[:, 0]).at[5:, 1].set(mlp2_w2[:, 0])
    b2cat = jnp.concatenate([mlp1_b2, mlp2_b2])[:, None]

    wspec = pl.BlockSpec((1, EB), lambda i: (0, i))
    full = lambda shp: pl.BlockSpec(shp, lambda i: tuple(0 for _ in shp))
    w1e, w2e = pl.pallas_call(
        _edge_w_body,
        grid=(EPAD // EB,),
        in_specs=[pl.BlockSpec((EB, DE), lambda i: (i, 0)),
                  full((DE, 10)), full((10, 1)), full((10, 2)), full((2, 1))],
        out_specs=[wspec, wspec],
        out_shape=[jax.ShapeDtypeStruct((1, EPAD), f32),
                   jax.ShapeDtypeStruct((1, EPAD), f32)],
    )(edge_attr, wcat, bcat, w2cat, b2cat)
    w1e = w1e.reshape(EPAD)
    w2e = w2e.reshape(EPAD)

    sc1 = pl.kernel(
        _sc_msg1_body,
        out_type=[jax.ShapeDtypeStruct((NC * 5 * NPAD,), f32),
                  jax.ShapeDtypeStruct((NPAD,), f32),
                  jax.ShapeDtypeStruct((NPAD,), f32),
                  jax.ShapeDtypeStruct((EPAD,), f32)],
        mesh=_sc_mesh(),
        compiler_params=pltpu.CompilerParams(needs_layout_passes=False),
        scratch_types=[
            pltpu.VMEM((ECHUNK,), jnp.int32),
            pltpu.VMEM((ECHUNK,), jnp.int32),
            pltpu.VMEM((ECHUNK,), f32),
            pltpu.VMEM((ECHUNK,), f32),
            pltpu.VMEM((ECHUNK,), f32),
            pltpu.VMEM((ECHUNK,), f32),
            pltpu.VMEM((ECHUNK,), f32),
            pltpu.VMEM((NPAD,), f32),
            pltpu.VMEM((NPAD,), f32),
            pltpu.VMEM((5 * NPAD,), f32),
            pltpu.VMEM((NPS,), f32),
            pltpu.VMEM_SHARED((NPAD,), f32),
            pltpu.VMEM_SHARED((NPAD,), f32),
            pltpu.VMEM_SHARED((NPAD,), f32),
            pltpu.VMEM_SHARED((NPAD,), f32),
            pltpu.VMEM_SHARED((NPAD,), f32),
            pltpu.VMEM_SHARED((NPAD,), f32),
            pltpu.VMEM_SHARED((NPAD,), f32),
            pltpu.VMEM_SHARED((NPAD,), f32),
            pltpu.VMEM_SHARED((NPAD,), f32),
        ],
    )
    msg1, deg1, deg2, wn2 = sc1(row_p, col_p, w1e, w2e, xlt.reshape(5 * NPAD))
    msg1 = msg1.reshape(NC, 5, NPAD)
    deg1 = deg1[None, :]
    deg2 = deg2[None, :]

    xl2 = pl.pallas_call(
        _mid_body,
        in_specs=[
            _full_spec((NC, 5, NPAD)),
            _full_spec((1, NPAD)),
            _full_spec((5, NPAD)),
            _full_spec((1, NPAD)),
            _full_spec((5, 1)),
            _full_spec((5, 1)),
            _full_spec((5, 1)),
            _full_spec((5, 1)),
            _full_spec((5, 1)),
        ],
        out_specs=_full_spec((1, NPAD)),
        out_shape=jax.ShapeDtypeStruct((1, NPAD), f32),
    )(msg1, deg1, xlt, batch_p, b1[:, None], gn_w[:, None],
      gn_b[:, None], gn_ms[:, None], W2)

    sc2 = pl.kernel(
        _sc_msg2_body,
        out_type=[jax.ShapeDtypeStruct((NC * NPAD,), f32)],
        mesh=_sc_mesh(),
        compiler_params=pltpu.CompilerParams(needs_layout_passes=False),
        scratch_types=[
            pltpu.VMEM((ECHUNK,), jnp.int32),
            pltpu.VMEM((ECHUNK,), jnp.int32),
            pltpu.VMEM((ECHUNK,), f32),
            pltpu.VMEM((ECHUNK,), f32),
            pltpu.VMEM((NPAD,), f32),
            pltpu.VMEM((NPS,), f32),
            pltpu.VMEM_SHARED((NPAD,), f32),
        ],
    )
    (msg2,) = sc2(row_p, col_p, wn2, xl2.reshape(NPAD))
    msg2 = msg2.reshape(NC, NPAD)

    out = pl.pallas_call(
        _final_body,
        in_specs=[_full_spec((NC, NPAD)), _full_spec((1, NPAD)),
                  _full_spec((1, NPAD)), _full_spec((1, 1))],
        out_specs=_full_spec((1, NPAD)),
        out_shape=jax.ShapeDtypeStruct((1, NPAD), f32),
    )(msg2, deg2, xl2, b2[:, None])

    return out[0, :N, None]


# speedup vs baseline: 1.1725x; 1.1175x over previous
import jax
import jax.numpy as jnp
from jax import lax
from jax.experimental import pallas as pl
from jax.experimental.pallas import tpu as pltpu
from jax.experimental.pallas import tpu_sc as plsc

N = 10000
E = 160000
D = 256
DE = 16
G = 64

NPAD = 10240
EPAD = 163840
NC = 2
NS = 16
NW = NC * NS
NPS = NPAD // NS
ECHUNK = EPAD // NW
EB = 16384
XB = 1024

_HIGH = lax.Precision.HIGHEST


def _xl1_body(x_ref, w_ref, o_ref):
    i = pl.program_id(0)
    n_glob = i * XB + lax.broadcasted_iota(jnp.int32, (1, XB), 1)
    o = lax.dot_general(w_ref[...], x_ref[...],
                        (((0,), (1,)), ((), ())),
                        precision=_HIGH)
    o_ref[...] = jnp.where(n_glob < N, o, 0.0)


EB_LAST = E - (EPAD // EB - 1) * EB


def _edge_w_body(ea_ref, wcat, bcat, w2cat, b2cat, w1_ref, w2_ref):
    i = pl.program_id(0)
    e_glob = i * EB + lax.broadcasted_iota(jnp.int32, (1, EB), 1)
    valid = e_glob < E
    h = lax.dot_general(wcat[...], ea_ref[...], (((0,), (1,)), ((), ())))
    h = jnp.maximum(h + bcat[...], 0.0)
    o = lax.dot_general(w2cat[...], h, (((0,), (0,)), ((), ())))
    o = jax.nn.sigmoid(o + b2cat[...])
    w1_ref[...] = jnp.where(valid, o[0:1, :], 0.0)
    w2_ref[...] = jnp.where(valid, o[1:2, :], 0.0)


def _mid_body(msg_ref, deg_ref, xlt_ref, batch_ref, b1_ref, gnw_ref, gnb_ref,
              gnms_ref, w2_ref, out_ref):
    h = (msg_ref[0] + msg_ref[1]
         + xlt_ref[...] / deg_ref[...]
         + b1_ref[...])
    ohT = (lax.broadcasted_iota(jnp.int32, (G, NPAD), 0)
           == batch_ref[...]).astype(jnp.float32)
    cnt = jnp.maximum(jnp.sum(ohT, axis=1), 1.0)[None, :]
    seg = lax.dot_general(h, ohT, (((1,), (1,)), ((), ())),
                          precision=_HIGH)
    mean = seg / cnt
    mean_b = lax.dot_general(mean, ohT, (((1,), (0,)), ((), ())),
                             precision=_HIGH)
    out = h - mean_b * gnms_ref[...]
    var = lax.dot_general(out * out, ohT, (((1,), (1,)), ((), ())),
                          precision=_HIGH) / cnt
    std = jnp.sqrt(var + 1e-5)
    std_b = lax.dot_general(std, ohT, (((1,), (0,)), ((), ())),
                            precision=_HIGH)
    std_b = jnp.where(std_b > 0.0, std_b, 1.0)
    hn = gnw_ref[...] * out / std_b + gnb_ref[...]
    hr = jnp.maximum(hn, 0.0)
    out_ref[...] = lax.dot_general(w2_ref[...], hr, (((0,), (0,)), ((), ())),
                                   precision=_HIGH)


def _final_body(msg_ref, deg_ref, xl2_ref, b2_ref, out_ref):
    h = (msg_ref[pl.ds(0, 1), :] + msg_ref[pl.ds(1, 1), :]
         + xl2_ref[...] / deg_ref[...] + b2_ref[...])
    out_ref[...] = jax.nn.sigmoid(h)


def _rsqrt16(x):
    i = plsc.bitcast(x, jnp.int32)
    i = jnp.int32(0x5F3759DF) - lax.shift_right_logical(i, 1)
    y = plsc.bitcast(i, jnp.float32)
    hx = 0.5 * x
    for _ in range(4):
        y = y * (1.5 - (hx * y) * y)
    return y


def _sc_deg_body(col_h, w1_h, w2_h,
                 deg1_o, deg2_o,
                 colb, w1b, w2b, nodeb,
                 deg1s, deg2s):
    c = lax.axis_index("c")
    s = lax.axis_index("s")
    wid = c * NS + s
    nbase = s * NPS

    init = jnp.where(c == 0, 1.0, 0.0)

    def init_loop(i, _):
        nodeb[pl.ds(i * 16, 16)] = jnp.full((16,), init, jnp.float32)
        return 0
    lax.fori_loop(0, NPS // 16, init_loop, 0)
    pltpu.sync_copy(nodeb, deg1s.at[pl.ds(nbase, NPS)])
    pltpu.sync_copy(nodeb, deg2s.at[pl.ds(nbase, NPS)])
    plsc.subcore_barrier()

    ebase = wid * ECHUNK
    pltpu.sync_copy(col_h.at[pl.ds(ebase, ECHUNK)], colb)
    pltpu.sync_copy(w1_h.at[pl.ds(ebase, ECHUNK)], w1b)
    pltpu.sync_copy(w1b, deg1s.at[colb], add=True)
    pltpu.sync_copy(w2_h.at[pl.ds(ebase, ECHUNK)], w2b)
    pltpu.sync_copy(w2b, deg2s.at[colb], add=True)
    plsc.subcore_barrier()

    pltpu.sync_copy(deg1s.at[pl.ds(nbase, NPS)],
                    deg1_o.at[pl.ds(c * NPAD + nbase, NPS)])
    pltpu.sync_copy(deg2s.at[pl.ds(nbase, NPS)],
                    deg2_o.at[pl.ds(c * NPAD + nbase, NPS)])


def _sc_msg1_body(row_h, col_h, w1_h, w2_h, xlt_h, deg1p_h, deg2p_h,
                  msg_o, deg1_o, deg2_o, wn2_o,
                  rowb, colb, w1b, w2b, wn1b, wn2b, valb,
                  dis1l, dis2l, xll, nodeb, nodeb2,
                  dis1s, dis2s, m0s, m1s, m2s, m3s, m4s):
    c = lax.axis_index("c")
    s = lax.axis_index("s")
    wid = c * NS + s
    nbase = s * NPS
    msgs = (m0s, m1s, m2s, m3s, m4s)

    def zero_loop(i, _):
        nodeb[pl.ds(i * 16, 16)] = jnp.zeros((16,), jnp.float32)
        return 0
    lax.fori_loop(0, NPS // 16, zero_loop, 0)
    for m in msgs:
        pltpu.sync_copy(nodeb, m.at[pl.ds(nbase, NPS)])

    for pref, dego, diss in ((deg1p_h, deg1_o, dis1s),
                             (deg2p_h, deg2_o, dis2s)):
        pltpu.sync_copy(pref.at[pl.ds(nbase, NPS)], nodeb)
        pltpu.sync_copy(pref.at[pl.ds(NPAD + nbase, NPS)], nodeb2)

        def comb_loop(i, _):
            sl = pl.ds(i * 16, 16)
            d = nodeb[sl] + nodeb2[sl]
            nodeb[sl] = d
            nodeb2[sl] = _rsqrt16(d)
            return 0
        lax.fori_loop(0, NPS // 16, comb_loop, 0)

        @pl.when(c == 0)
        def _():
            pltpu.sync_copy(nodeb, dego.at[pl.ds(nbase, NPS)])
        pltpu.sync_copy(nodeb2, diss.at[pl.ds(nbase, NPS)])
    plsc.subcore_barrier()

    pltpu.sync_copy(dis1s, dis1l)
    pltpu.sync_copy(dis2s, dis2l)
    pltpu.sync_copy(xlt_h, xll)

    ebase = wid * ECHUNK
    pltpu.sync_copy(row_h.at[pl.ds(ebase, ECHUNK)], rowb)
    pltpu.sync_copy(col_h.at[pl.ds(ebase, ECHUNK)], colb)
    pltpu.sync_copy(w1_h.at[pl.ds(ebase, ECHUNK)], w1b)
    pltpu.sync_copy(w2_h.at[pl.ds(ebase, ECHUNK)], w2b)

    def wn_loop(i, _):
        sl = pl.ds(i * 16, 16)
        r = rowb[sl]
        cc = colb[sl]
        wn1b[sl] = (w1b[sl] * plsc.load_gather(dis1l, [r])) \
            * plsc.load_gather(dis1l, [cc])
        wn2b[sl] = (w2b[sl] * plsc.load_gather(dis2l, [r])) \
            * plsc.load_gather(dis2l, [cc])
        return 0
    lax.fori_loop(0, ECHUNK // 16, wn_loop, 0)
    pltpu.sync_copy(wn2b, wn2_o.at[pl.ds(ebase, ECHUNK)])

    for f in range(5):
        foff = jnp.int32(f * NPAD)

        def msg_loop(i, _):
            sl = pl.ds(i * 16, 16)
            valb[sl] = wn1b[sl] * plsc.load_gather(xll, [rowb[sl] + foff])
            return 0
        lax.fori_loop(0, ECHUNK // 16, msg_loop, 0)
        pltpu.sync_copy(valb, msgs[f].at[colb], add=True)
    plsc.subcore_barrier()

    for f in range(5):
        moff = (c * 5 + f) * NPAD + nbase
        pltpu.sync_copy(msgs[f].at[pl.ds(nbase, NPS)],
                        msg_o.at[pl.ds(moff, NPS)])


def _sc_msg2_body(row_h, col_h, wn_h, xl2_h,
                  msg_o,
                  rowb, colb, wnb, valb, xl2l, nodeb,
                  m0s):
    c = lax.axis_index("c")
    s = lax.axis_index("s")
    wid = c * NS + s
    nbase = s * NPS

    def zero_loop(i, _):
        nodeb[pl.ds(i * 16, 16)] = jnp.zeros((16,), jnp.float32)
        return 0
    lax.fori_loop(0, NPS // 16, zero_loop, 0)
    pltpu.sync_copy(nodeb, m0s.at[pl.ds(nbase, NPS)])
    plsc.subcore_barrier()

    pltpu.sync_copy(xl2_h, xl2l)
    ebase = wid * ECHUNK
    pltpu.sync_copy(row_h.at[pl.ds(ebase, ECHUNK)], rowb)
    pltpu.sync_copy(col_h.at[pl.ds(ebase, ECHUNK)], colb)
    pltpu.sync_copy(wn_h.at[pl.ds(ebase, ECHUNK)], wnb)

    def msg_loop(i, _):
        sl = pl.ds(i * 16, 16)
        valb[sl] = wnb[sl] * plsc.load_gather(xl2l, [rowb[sl]])
        return 0
    lax.fori_loop(0, ECHUNK // 16, msg_loop, 0)
    pltpu.sync_copy(valb, m0s.at[colb], add=True)
    plsc.subcore_barrier()

    pltpu.sync_copy(m0s.at[pl.ds(nbase, NPS)],
                    msg_o.at[pl.ds(c * NPAD + nbase, NPS)])


def _sc_mesh():
    return plsc.VectorSubcoreMesh(core_axis_name="c", subcore_axis_name="s",
                                  num_cores=NC, num_subcores=NS)


_full_spec = lambda shp: pl.BlockSpec(shp, lambda: tuple(0 for _ in shp))


@jax.jit
def kernel(x, edge_index, edge_attr, batch_idx, W1, b1, mlp1_w1, mlp1_b1,
           mlp1_w2, mlp1_b2, gn_w, gn_b, gn_ms, W2, b2, mlp2_w1, mlp2_b1,
           mlp2_w2, mlp2_b2):
    f32 = jnp.float32

    row_p = jnp.pad(edge_index[0], (0, EPAD - E), constant_values=NPAD - 1)
    col_p = jnp.pad(edge_index[1], (0, EPAD - E), constant_values=NPAD - 1)
    batch_p = jnp.pad(batch_idx, (0, NPAD - N), constant_values=-1)[None, :]

    xlt = pl.pallas_call(
        _xl1_body,
        grid=(NPAD // XB,),
        in_specs=[pl.BlockSpec((XB, D), lambda i: (i, 0)),
                  pl.BlockSpec((D, 5), lambda i: (0, 0))],
        out_specs=pl.BlockSpec((5, XB), lambda i: (0, i)),
        out_shape=jax.ShapeDtypeStruct((5, NPAD), f32),
    )(x, W1)

    wcat = jnp.concatenate([mlp1_w1, mlp2_w1], axis=1)
    bcat = jnp.concatenate([mlp1_b1, mlp2_b1])[:, None]
    w2cat = jnp.zeros((10, 2), f32)
    w2cat = w2cat.at[:5, 0].set(mlp1_w2[:, 0]).at[5:, 1].set(mlp2_w2[:, 0])
    b2cat = jnp.concatenate([mlp1_b2, mlp2_b2])[:, None]

    wspec = pl.BlockSpec((1, EB), lambda i: (0, i))
    full = lambda shp: pl.BlockSpec(shp, lambda i: tuple(0 for _ in shp))
    w1e, w2e = pl.pallas_call(
        _edge_w_body,
        grid=(EPAD // EB,),
        in_specs=[pl.BlockSpec((EB, DE), lambda i: (i, 0)),
                  full((DE, 10)), full((10, 1)), full((10, 2)), full((2, 1))],
        out_specs=[wspec, wspec],
        out_shape=[jax.ShapeDtypeStruct((1, EPAD), f32),
                   jax.ShapeDtypeStruct((1, EPAD), f32)],
    )(edge_attr, wcat, bcat, w2cat, b2cat)
    w1e = w1e.reshape(EPAD)
    w2e = w2e.reshape(EPAD)

    sc_deg = pl.kernel(
        _sc_deg_body,
        out_type=[jax.ShapeDtypeStruct((NC * NPAD,), f32),
                  jax.ShapeDtypeStruct((NC * NPAD,), f32)],
        mesh=_sc_mesh(),
        compiler_params=pltpu.CompilerParams(needs_layout_passes=False),
        scratch_types=[
            pltpu.VMEM((ECHUNK,), jnp.int32),
            pltpu.VMEM((ECHUNK,), f32),
            pltpu.VMEM((ECHUNK,), f32),
            pltpu.VMEM((NPS,), f32),
            pltpu.VMEM_SHARED((NPAD,), f32),
            pltpu.VMEM_SHARED((NPAD,), f32),
        ],
    )
    deg1p, deg2p = sc_deg(col_p, w1e, w2e)

    sc1 = pl.kernel(
        _sc_msg1_body,
        out_type=[jax.ShapeDtypeStruct((NC * 5 * NPAD,), f32),
                  jax.ShapeDtypeStruct((NPAD,), f32),
                  jax.ShapeDtypeStruct((NPAD,), f32),
                  jax.ShapeDtypeStruct((EPAD,), f32)],
        mesh=_sc_mesh(),
        compiler_params=pltpu.CompilerParams(needs_layout_passes=False),
        scratch_types=[
            pltpu.VMEM((ECHUNK,), jnp.int32),
            pltpu.VMEM((ECHUNK,), jnp.int32),
            pltpu.VMEM((ECHUNK,), f32),
            pltpu.VMEM((ECHUNK,), f32),
            pltpu.VMEM((ECHUNK,), f32),
            pltpu.VMEM((ECHUNK,), f32),
            pltpu.VMEM((ECHUNK,), f32),
            pltpu.VMEM((NPAD,), f32),
            pltpu.VMEM((NPAD,), f32),
            pltpu.VMEM((5 * NPAD,), f32),
            pltpu.VMEM((NPS,), f32),
            pltpu.VMEM((NPS,), f32),
            pltpu.VMEM_SHARED((NPAD,), f32),
            pltpu.VMEM_SHARED((NPAD,), f32),
            pltpu.VMEM_SHARED((NPAD,), f32),
            pltpu.VMEM_SHARED((NPAD,), f32),
            pltpu.VMEM_SHARED((NPAD,), f32),
            pltpu.VMEM_SHARED((NPAD,), f32),
            pltpu.VMEM_SHARED((NPAD,), f32),
        ],
    )
    msg1, deg1, deg2, wn2 = sc1(row_p, col_p, w1e, w2e,
                                xlt.reshape(5 * NPAD), deg1p, deg2p)
    msg1 = msg1.reshape(NC, 5, NPAD)
    deg1 = deg1[None, :]
    deg2 = deg2[None, :]

    xl2 = pl.pallas_call(
        _mid_body,
        in_specs=[
            _full_spec((NC, 5, NPAD)),
            _full_spec((1, NPAD)),
            _full_spec((5, NPAD)),
            _full_spec((1, NPAD)),
            _full_spec((5, 1)),
            _full_spec((5, 1)),
            _full_spec((5, 1)),
            _full_spec((5, 1)),
            _full_spec((5, 1)),
        ],
        out_specs=_full_spec((1, NPAD)),
        out_shape=jax.ShapeDtypeStruct((1, NPAD), f32),
    )(msg1, deg1, xlt, batch_p, b1[:, None], gn_w[:, None],
      gn_b[:, None], gn_ms[:, None], W2)

    sc2 = pl.kernel(
        _sc_msg2_body,
        out_type=[jax.ShapeDtypeStruct((NC * NPAD,), f32)],
        mesh=_sc_mesh(),
        compiler_params=pltpu.CompilerParams(needs_layout_passes=False),
        scratch_types=[
            pltpu.VMEM((ECHUNK,), jnp.int32),
            pltpu.VMEM((ECHUNK,), jnp.int32),
            pltpu.VMEM((ECHUNK,), f32),
            pltpu.VMEM((ECHUNK,), f32),
            pltpu.VMEM((NPAD,), f32),
            pltpu.VMEM((NPS,), f32),
            pltpu.VMEM_SHARED((NPAD,), f32),
        ],
    )
    (msg2,) = sc2(row_p, col_p, wn2, xl2.reshape(NPAD))
    msg2 = msg2.reshape(NC, NPAD)

    out = pl.pallas_call(
        _final_body,
        in_specs=[_full_spec((NC, NPAD)), _full_spec((1, NPAD)),
                  _full_spec((1, NPAD)), _full_spec((1, 1))],
        out_specs=_full_spec((1, NPAD)),
        out_shape=jax.ShapeDtypeStruct((1, NPAD), f32),
    )(msg2, deg2, xl2, b2[:, None])

    return out[0, :N, None]


# speedup vs baseline: 1.1757x; 1.0027x over previous
import jax
import jax.numpy as jnp
from jax import lax
from jax.experimental import pallas as pl
from jax.experimental.pallas import tpu as pltpu
from jax.experimental.pallas import tpu_sc as plsc

N = 10000
E = 160000
D = 256
DE = 16
G = 64

NPAD = 10240
EPAD = 163840
NC = 2
NS = 16
NW = NC * NS
NPS = NPAD // NS
ECHUNK = EPAD // NW
EB = 16384
XB = 1024

_HIGH = lax.Precision.HIGHEST


def _xl1_body(x_ref, w_ref, o_ref):
    i = pl.program_id(0)
    n_glob = i * XB + lax.broadcasted_iota(jnp.int32, (1, XB), 1)
    o = lax.dot_general(w_ref[...], x_ref[...],
                        (((0,), (1,)), ((), ())),
                        precision=_HIGH)
    o_ref[...] = jnp.where(n_glob < N, o, 0.0)


EB_LAST = E - (EPAD // EB - 1) * EB


def _edge_w_body(ea_ref, wcat, bcat, w2cat, b2cat, w1_ref, w2_ref):
    i = pl.program_id(0)
    e_glob = i * EB + lax.broadcasted_iota(jnp.int32, (1, EB), 1)
    valid = e_glob < E
    h = lax.dot_general(wcat[...], ea_ref[...], (((0,), (1,)), ((), ())))
    h = jnp.maximum(h + bcat[...], 0.0)
    o = lax.dot_general(w2cat[...], h, (((0,), (0,)), ((), ())))
    o = jax.nn.sigmoid(o + b2cat[...])
    w1_ref[...] = jnp.where(valid, o[0:1, :], 0.0)
    w2_ref[...] = jnp.where(valid, o[1:2, :], 0.0)


def _mid_body(msg_ref, deg_ref, xlt_ref, batch_ref, b1_ref, gnw_ref, gnb_ref,
              gnms_ref, w2_ref, out_ref):
    h = ((msg_ref[0] + msg_ref[1]) * lax.rsqrt(deg_ref[...])
         + xlt_ref[...] / deg_ref[...]
         + b1_ref[...])
    ohT = (lax.broadcasted_iota(jnp.int32, (G, NPAD), 0)
           == batch_ref[...]).astype(jnp.float32)
    cnt = jnp.maximum(jnp.sum(ohT, axis=1), 1.0)[None, :]
    seg = lax.dot_general(h, ohT, (((1,), (1,)), ((), ())),
                          precision=_HIGH)
    mean = seg / cnt
    mean_b = lax.dot_general(mean, ohT, (((1,), (0,)), ((), ())),
                             precision=_HIGH)
    out = h - mean_b * gnms_ref[...]
    var = lax.dot_general(out * out, ohT, (((1,), (1,)), ((), ())),
                          precision=_HIGH) / cnt
    std = jnp.sqrt(var + 1e-5)
    std_b = lax.dot_general(std, ohT, (((1,), (0,)), ((), ())),
                            precision=_HIGH)
    std_b = jnp.where(std_b > 0.0, std_b, 1.0)
    hn = gnw_ref[...] * out / std_b + gnb_ref[...]
    hr = jnp.maximum(hn, 0.0)
    out_ref[...] = lax.dot_general(w2_ref[...], hr, (((0,), (0,)), ((), ())),
                                   precision=_HIGH)


def _final_body(msg_ref, deg_ref, xl2_ref, b2_ref, out_ref):
    h = ((msg_ref[pl.ds(0, 1), :] + msg_ref[pl.ds(1, 1), :])
         * lax.rsqrt(deg_ref[...])
         + xl2_ref[...] / deg_ref[...] + b2_ref[...])
    out_ref[...] = jax.nn.sigmoid(h)


def _rsqrt16(x):
    i = plsc.bitcast(x, jnp.int32)
    i = jnp.int32(0x5F3759DF) - lax.shift_right_logical(i, 1)
    y = plsc.bitcast(i, jnp.float32)
    hx = 0.5 * x
    for _ in range(4):
        y = y * (1.5 - (hx * y) * y)
    return y


def _sc_deg_body(col_h, w1_h, w2_h,
                 deg1_o, deg2_o,
                 colb, w1b, w2b, nodeb,
                 deg1s, deg2s):
    c = lax.axis_index("c")
    s = lax.axis_index("s")
    wid = c * NS + s
    nbase = s * NPS

    init = jnp.where(c == 0, 1.0, 0.0)

    def init_loop(i, _):
        nodeb[pl.ds(i * 16, 16)] = jnp.full((16,), init, jnp.float32)
        return 0
    lax.fori_loop(0, NPS // 16, init_loop, 0)
    pltpu.sync_copy(nodeb, deg1s.at[pl.ds(nbase, NPS)])
    pltpu.sync_copy(nodeb, deg2s.at[pl.ds(nbase, NPS)])
    plsc.subcore_barrier()

    ebase = wid * ECHUNK
    pltpu.sync_copy(col_h.at[pl.ds(ebase, ECHUNK)], colb)
    pltpu.sync_copy(w1_h.at[pl.ds(ebase, ECHUNK)], w1b)
    pltpu.sync_copy(w1b, deg1s.at[colb], add=True)
    pltpu.sync_copy(w2_h.at[pl.ds(ebase, ECHUNK)], w2b)
    pltpu.sync_copy(w2b, deg2s.at[colb], add=True)
    plsc.subcore_barrier()

    pltpu.sync_copy(deg1s.at[pl.ds(nbase, NPS)],
                    deg1_o.at[pl.ds(c * NPAD + nbase, NPS)])
    pltpu.sync_copy(deg2s.at[pl.ds(nbase, NPS)],
                    deg2_o.at[pl.ds(c * NPAD + nbase, NPS)])


def _sc_msg1_body(row_h, col_h, w1_h, w2_h, xlt_h, deg1p_h, deg2p_h,
                  msg_o, deg1_o, deg2_o, wn2_o,
                  rowb, colb, w1b, w2b, wn1b, wn2b, valb,
                  dis1l, dis2l, xll, nodeb, nodeb2,
                  dis1s, dis2s, m0s, m1s, m2s, m3s, m4s):
    c = lax.axis_index("c")
    s = lax.axis_index("s")
    wid = c * NS + s
    nbase = s * NPS
    msgs = (m0s, m1s, m2s, m3s, m4s)

    def zero_loop(i, _):
        nodeb[pl.ds(i * 16, 16)] = jnp.zeros((16,), jnp.float32)
        return 0
    lax.fori_loop(0, NPS // 16, zero_loop, 0)
    for m in msgs:
        pltpu.sync_copy(nodeb, m.at[pl.ds(nbase, NPS)])

    for pref, dego, diss in ((deg1p_h, deg1_o, dis1s),
                             (deg2p_h, deg2_o, dis2s)):
        pltpu.sync_copy(pref.at[pl.ds(nbase, NPS)], nodeb)
        pltpu.sync_copy(pref.at[pl.ds(NPAD + nbase, NPS)], nodeb2)

        def comb_loop(i, _):
            sl = pl.ds(i * 16, 16)
            d = nodeb[sl] + nodeb2[sl]
            nodeb[sl] = d
            nodeb2[sl] = _rsqrt16(d)
            return 0
        lax.fori_loop(0, NPS // 16, comb_loop, 0)

        @pl.when(c == 0)
        def _():
            pltpu.sync_copy(nodeb, dego.at[pl.ds(nbase, NPS)])
        pltpu.sync_copy(nodeb2, diss.at[pl.ds(nbase, NPS)])
    plsc.subcore_barrier()

    pltpu.sync_copy(dis1s, dis1l)
    pltpu.sync_copy(dis2s, dis2l)
    pltpu.sync_copy(xlt_h, xll)

    ebase = wid * ECHUNK
    pltpu.sync_copy(row_h.at[pl.ds(ebase, ECHUNK)], rowb)
    pltpu.sync_copy(col_h.at[pl.ds(ebase, ECHUNK)], colb)
    pltpu.sync_copy(w1_h.at[pl.ds(ebase, ECHUNK)], w1b)
    pltpu.sync_copy(w2_h.at[pl.ds(ebase, ECHUNK)], w2b)

    def wn_loop(i, _):
        sl = pl.ds(i * 16, 16)
        r = rowb[sl]
        wn1b[sl] = w1b[sl] * plsc.load_gather(dis1l, [r])
        wn2b[sl] = w2b[sl] * plsc.load_gather(dis2l, [r])
        return 0
    lax.fori_loop(0, ECHUNK // 16, wn_loop, 0)
    pltpu.sync_copy(wn2b, wn2_o.at[pl.ds(ebase, ECHUNK)])

    for f in range(5):
        foff = jnp.int32(f * NPAD)

        def msg_loop(i, _):
            sl = pl.ds(i * 16, 16)
            valb[sl] = wn1b[sl] * plsc.load_gather(xll, [rowb[sl] + foff])
            return 0
        lax.fori_loop(0, ECHUNK // 16, msg_loop, 0)
        pltpu.sync_copy(valb, msgs[f].at[colb], add=True)
    plsc.subcore_barrier()

    for f in range(5):
        moff = (c * 5 + f) * NPAD + nbase
        pltpu.sync_copy(msgs[f].at[pl.ds(nbase, NPS)],
                        msg_o.at[pl.ds(moff, NPS)])


def _sc_msg2_body(row_h, col_h, wn_h, xl2_h,
                  msg_o,
                  rowb, colb, wnb, valb, xl2l, nodeb,
                  m0s):
    c = lax.axis_index("c")
    s = lax.axis_index("s")
    wid = c * NS + s
    nbase = s * NPS

    def zero_loop(i, _):
        nodeb[pl.ds(i * 16, 16)] = jnp.zeros((16,), jnp.float32)
        return 0
    lax.fori_loop(0, NPS // 16, zero_loop, 0)
    pltpu.sync_copy(nodeb, m0s.at[pl.ds(nbase, NPS)])
    plsc.subcore_barrier()

    pltpu.sync_copy(xl2_h, xl2l)
    ebase = wid * ECHUNK
    pltpu.sync_copy(row_h.at[pl.ds(ebase, ECHUNK)], rowb)
    pltpu.sync_copy(col_h.at[pl.ds(ebase, ECHUNK)], colb)
    pltpu.sync_copy(wn_h.at[pl.ds(ebase, ECHUNK)], wnb)

    def msg_loop(i, _):
        sl = pl.ds(i * 16, 16)
        valb[sl] = wnb[sl] * plsc.load_gather(xl2l, [rowb[sl]])
        return 0
    lax.fori_loop(0, ECHUNK // 16, msg_loop, 0)
    pltpu.sync_copy(valb, m0s.at[colb], add=True)
    plsc.subcore_barrier()

    pltpu.sync_copy(m0s.at[pl.ds(nbase, NPS)],
                    msg_o.at[pl.ds(c * NPAD + nbase, NPS)])


def _sc_mesh():
    return plsc.VectorSubcoreMesh(core_axis_name="c", subcore_axis_name="s",
                                  num_cores=NC, num_subcores=NS)


_full_spec = lambda shp: pl.BlockSpec(shp, lambda: tuple(0 for _ in shp))


@jax.jit
def kernel(x, edge_index, edge_attr, batch_idx, W1, b1, mlp1_w1, mlp1_b1,
           mlp1_w2, mlp1_b2, gn_w, gn_b, gn_ms, W2, b2, mlp2_w1, mlp2_b1,
           mlp2_w2, mlp2_b2):
    f32 = jnp.float32

    row_p = jnp.pad(edge_index[0], (0, EPAD - E), constant_values=NPAD - 1)
    col_p = jnp.pad(edge_index[1], (0, EPAD - E), constant_values=NPAD - 1)
    batch_p = jnp.pad(batch_idx, (0, NPAD - N), constant_values=-1)[None, :]

    xlt = pl.pallas_call(
        _xl1_body,
        grid=(NPAD // XB,),
        in_specs=[pl.BlockSpec((XB, D), lambda i: (i, 0)),
                  pl.BlockSpec((D, 5), lambda i: (0, 0))],
        out_specs=pl.BlockSpec((5, XB), lambda i: (0, i)),
        out_shape=jax.ShapeDtypeStruct((5, NPAD), f32),
    )(x, W1)

    wcat = jnp.concatenate([mlp1_w1, mlp2_w1], axis=1)
    bcat = jnp.concatenate([mlp1_b1, mlp2_b1])[:, None]
    w2cat = jnp.zeros((10, 2), f32)
    w2cat = w2cat.at[:5, 0].set(mlp1_w2[:, 0]).at[5:, 1].set(mlp2_w2[:, 0])
    b2cat = jnp.concatenate([mlp1_b2, mlp2_b2])[:, None]

    wspec = pl.BlockSpec((1, EB), lambda i: (0, i))
    full = lambda shp: pl.BlockSpec(shp, lambda i: tuple(0 for _ in shp))
    w1e, w2e = pl.pallas_call(
        _edge_w_body,
        grid=(EPAD // EB,),
        in_specs=[pl.BlockSpec((EB, DE), lambda i: (i, 0)),
                  full((DE, 10)), full((10, 1)), full((10, 2)), full((2, 1))],
        out_specs=[wspec, wspec],
        out_shape=[jax.ShapeDtypeStruct((1, EPAD), f32),
                   jax.ShapeDtypeStruct((1, EPAD), f32)],
    )(edge_attr, wcat, bcat, w2cat, b2cat)
    w1e = w1e.reshape(EPAD)
    w2e = w2e.reshape(EPAD)

    sc_deg = pl.kernel(
        _sc_deg_body,
        out_type=[jax.ShapeDtypeStruct((NC * NPAD,), f32),
                  jax.ShapeDtypeStruct((NC * NPAD,), f32)],
        mesh=_sc_mesh(),
        compiler_params=pltpu.CompilerParams(needs_layout_passes=False),
        scratch_types=[
            pltpu.VMEM((ECHUNK,), jnp.int32),
            pltpu.VMEM((ECHUNK,), f32),
            pltpu.VMEM((ECHUNK,), f32),
            pltpu.VMEM((NPS,), f32),
            pltpu.VMEM_SHARED((NPAD,), f32),
            pltpu.VMEM_SHARED((NPAD,), f32),
        ],
    )
    deg1p, deg2p = sc_deg(col_p, w1e, w2e)

    sc1 = pl.kernel(
        _sc_msg1_body,
        out_type=[jax.ShapeDtypeStruct((NC * 5 * NPAD,), f32),
                  jax.ShapeDtypeStruct((NPAD,), f32),
                  jax.ShapeDtypeStruct((NPAD,), f32),
                  jax.ShapeDtypeStruct((EPAD,), f32)],
        mesh=_sc_mesh(),
        compiler_params=pltpu.CompilerParams(needs_layout_passes=False),
        scratch_types=[
            pltpu.VMEM((ECHUNK,), jnp.int32),
            pltpu.VMEM((ECHUNK,), jnp.int32),
            pltpu.VMEM((ECHUNK,), f32),
            pltpu.VMEM((ECHUNK,), f32),
            pltpu.VMEM((ECHUNK,), f32),
            pltpu.VMEM((ECHUNK,), f32),
            pltpu.VMEM((ECHUNK,), f32),
            pltpu.VMEM((NPAD,), f32),
            pltpu.VMEM((NPAD,), f32),
            pltpu.VMEM((5 * NPAD,), f32),
            pltpu.VMEM((NPS,), f32),
            pltpu.VMEM((NPS,), f32),
            pltpu.VMEM_SHARED((NPAD,), f32),
            pltpu.VMEM_SHARED((NPAD,), f32),
            pltpu.VMEM_SHARED((NPAD,), f32),
            pltpu.VMEM_SHARED((NPAD,), f32),
            pltpu.VMEM_SHARED((NPAD,), f32),
            pltpu.VMEM_SHARED((NPAD,), f32),
            pltpu.VMEM_SHARED((NPAD,), f32),
        ],
    )
    msg1, deg1, deg2, wn2 = sc1(row_p, col_p, w1e, w2e,
                                xlt.reshape(5 * NPAD), deg1p, deg2p)
    msg1 = msg1.reshape(NC, 5, NPAD)
    deg1 = deg1[None, :]
    deg2 = deg2[None, :]

    xl2 = pl.pallas_call(
        _mid_body,
        in_specs=[
            _full_spec((NC, 5, NPAD)),
            _full_spec((1, NPAD)),
            _full_spec((5, NPAD)),
            _full_spec((1, NPAD)),
            _full_spec((5, 1)),
            _full_spec((5, 1)),
            _full_spec((5, 1)),
            _full_spec((5, 1)),
            _full_spec((5, 1)),
        ],
        out_specs=_full_spec((1, NPAD)),
        out_shape=jax.ShapeDtypeStruct((1, NPAD), f32),
    )(msg1, deg1, xlt, batch_p, b1[:, None], gn_w[:, None],
      gn_b[:, None], gn_ms[:, None], W2)

    sc2 = pl.kernel(
        _sc_msg2_body,
        out_type=[jax.ShapeDtypeStruct((NC * NPAD,), f32)],
        mesh=_sc_mesh(),
        compiler_params=pltpu.CompilerParams(needs_layout_passes=False),
        scratch_types=[
            pltpu.VMEM((ECHUNK,), jnp.int32),
            pltpu.VMEM((ECHUNK,), jnp.int32),
            pltpu.VMEM((ECHUNK,), f32),
            pltpu.VMEM((ECHUNK,), f32),
            pltpu.VMEM((NPAD,), f32),
            pltpu.VMEM((NPS,), f32),
            pltpu.VMEM_SHARED((NPAD,), f32),
        ],
    )
    (msg2,) = sc2(row_p, col_p, wn2, xl2.reshape(NPAD))
    msg2 = msg2.reshape(NC, NPAD)

    out = pl.pallas_call(
        _final_body,
        in_specs=[_full_spec((NC, NPAD)), _full_spec((1, NPAD)),
                  _full_spec((1, NPAD)), _full_spec((1, 1))],
        out_specs=_full_spec((1, NPAD)),
        out_shape=jax.ShapeDtypeStruct((1, NPAD), f32),
    )(msg2, deg2, xl2, b2[:, None])

    return out[0, :N, None]


# speedup vs baseline: 1.2379x; 1.0529x over previous
import jax
import jax.numpy as jnp
from jax import lax
from jax.experimental import pallas as pl
from jax.experimental.pallas import tpu as pltpu
from jax.experimental.pallas import tpu_sc as plsc

N = 10000
E = 160000
D = 256
DE = 16
G = 64

NPAD = 10240
EPAD = 163840
NC = 2
NS = 16
NW = NC * NS
NPS = NPAD // NS
ECHUNK = EPAD // NW
EB = 16384
XB = 1024

_HIGH = lax.Precision.HIGHEST


def _xl1_body(x_ref, w_ref, o_ref):
    i = pl.program_id(0)
    n_glob = i * XB + lax.broadcasted_iota(jnp.int32, (1, XB), 1)
    o = lax.dot_general(w_ref[...], x_ref[...],
                        (((0,), (1,)), ((), ())),
                        precision=_HIGH)
    o_ref[...] = jnp.where(n_glob < N, o, 0.0)


EB_LAST = E - (EPAD // EB - 1) * EB


def _edge_w_body(ea_ref, wcat, bcat, w2cat, b2cat, w1_ref, w2_ref):
    i = pl.program_id(0)
    e_glob = i * EB + lax.broadcasted_iota(jnp.int32, (1, EB), 1)
    valid = e_glob < E
    h = lax.dot_general(wcat[...], ea_ref[...], (((0,), (1,)), ((), ())))
    h = jnp.maximum(h + bcat[...], 0.0)
    o = lax.dot_general(w2cat[...], h, (((0,), (0,)), ((), ())))
    o = jax.nn.sigmoid(o + b2cat[...])
    w1_ref[...] = jnp.where(valid, o[0:1, :], 0.0)
    w2_ref[...] = jnp.where(valid, o[1:2, :], 0.0)


def _mid_body(msg_ref, deg_ref, xlt_ref, batch_ref, b1_ref, gnw_ref, gnb_ref,
              gnms_ref, w2_ref, out_ref):
    h = ((msg_ref[0] + msg_ref[1]) * lax.rsqrt(deg_ref[...])
         + xlt_ref[...] / deg_ref[...]
         + b1_ref[...])
    ohT = (lax.broadcasted_iota(jnp.int32, (G, NPAD), 0)
           == batch_ref[...]).astype(jnp.float32)
    cnt = jnp.maximum(jnp.sum(ohT, axis=1), 1.0)[None, :]
    seg = lax.dot_general(h, ohT, (((1,), (1,)), ((), ())))
    mean = seg / cnt
    mean_b = lax.dot_general(mean, ohT, (((1,), (0,)), ((), ())))
    out = h - mean_b * gnms_ref[...]
    var = lax.dot_general(out * out, ohT, (((1,), (1,)), ((), ()))) / cnt
    std = jnp.sqrt(var + 1e-5)
    std_b = lax.dot_general(std, ohT, (((1,), (0,)), ((), ())))
    std_b = jnp.where(std_b > 0.0, std_b, 1.0)
    hn = gnw_ref[...] * out / std_b + gnb_ref[...]
    hr = jnp.maximum(hn, 0.0)
    out_ref[...] = lax.dot_general(w2_ref[...], hr, (((0,), (0,)), ((), ())))


def _final_body(msg_ref, deg_ref, xl2_ref, b2_ref, out_ref):
    h = ((msg_ref[pl.ds(0, 1), :] + msg_ref[pl.ds(1, 1), :])
         * lax.rsqrt(deg_ref[...])
         + xl2_ref[...] / deg_ref[...] + b2_ref[...])
    out_ref[...] = jax.nn.sigmoid(h)


def _rsqrt16(x):
    i = plsc.bitcast(x, jnp.int32)
    i = jnp.int32(0x5F3759DF) - lax.shift_right_logical(i, 1)
    y = plsc.bitcast(i, jnp.float32)
    hx = 0.5 * x
    for _ in range(4):
        y = y * (1.5 - (hx * y) * y)
    return y


def _sc_deg_body(col_h, w1_h, w2_h,
                 deg1_o, deg2_o,
                 colb, w1b, w2b, nodeb,
                 deg1s, deg2s):
    c = lax.axis_index("c")
    s = lax.axis_index("s")
    wid = c * NS + s
    nbase = s * NPS

    init = jnp.where(c == 0, 1.0, 0.0)

    def init_loop(i, _):
        nodeb[pl.ds(i * 16, 16)] = jnp.full((16,), init, jnp.float32)
        return 0
    lax.fori_loop(0, NPS // 16, init_loop, 0)
    pltpu.sync_copy(nodeb, deg1s.at[pl.ds(nbase, NPS)])
    pltpu.sync_copy(nodeb, deg2s.at[pl.ds(nbase, NPS)])
    plsc.subcore_barrier()

    ebase = wid * ECHUNK
    pltpu.sync_copy(col_h.at[pl.ds(ebase, ECHUNK)], colb)
    pltpu.sync_copy(w1_h.at[pl.ds(ebase, ECHUNK)], w1b)
    pltpu.sync_copy(w1b, deg1s.at[colb], add=True)
    pltpu.sync_copy(w2_h.at[pl.ds(ebase, ECHUNK)], w2b)
    pltpu.sync_copy(w2b, deg2s.at[colb], add=True)
    plsc.subcore_barrier()

    pltpu.sync_copy(deg1s.at[pl.ds(nbase, NPS)],
                    deg1_o.at[pl.ds(c * NPAD + nbase, NPS)])
    pltpu.sync_copy(deg2s.at[pl.ds(nbase, NPS)],
                    deg2_o.at[pl.ds(c * NPAD + nbase, NPS)])


def _sc_msg1_body(row_h, col_h, w1_h, w2_h, xlt_h, deg1p_h, deg2p_h,
                  msg_o, deg1_o, deg2_o, wn2_o,
                  rowb, colb, w1b, w2b, wn1b, wn2b, valb,
                  dis1l, dis2l, xll, nodeb, nodeb2,
                  dis1s, dis2s, m0s, m1s, m2s, m3s, m4s):
    c = lax.axis_index("c")
    s = lax.axis_index("s")
    wid = c * NS + s
    nbase = s * NPS
    msgs = (m0s, m1s, m2s, m3s, m4s)

    def zero_loop(i, _):
        nodeb[pl.ds(i * 16, 16)] = jnp.zeros((16,), jnp.float32)
        return 0
    lax.fori_loop(0, NPS // 16, zero_loop, 0)
    for m in msgs:
        pltpu.sync_copy(nodeb, m.at[pl.ds(nbase, NPS)])

    for pref, dego, diss in ((deg1p_h, deg1_o, dis1s),
                             (deg2p_h, deg2_o, dis2s)):
        pltpu.sync_copy(pref.at[pl.ds(nbase, NPS)], nodeb)
        pltpu.sync_copy(pref.at[pl.ds(NPAD + nbase, NPS)], nodeb2)

        def comb_loop(i, _):
            sl = pl.ds(i * 16, 16)
            d = nodeb[sl] + nodeb2[sl]
            nodeb[sl] = d
            nodeb2[sl] = _rsqrt16(d)
            return 0
        lax.fori_loop(0, NPS // 16, comb_loop, 0)

        @pl.when(c == 0)
        def _():
            pltpu.sync_copy(nodeb, dego.at[pl.ds(nbase, NPS)])
        pltpu.sync_copy(nodeb2, diss.at[pl.ds(nbase, NPS)])
    plsc.subcore_barrier()

    pltpu.sync_copy(dis1s, dis1l)
    pltpu.sync_copy(dis2s, dis2l)
    pltpu.sync_copy(xlt_h, xll)

    ebase = wid * ECHUNK
    pltpu.sync_copy(row_h.at[pl.ds(ebase, ECHUNK)], rowb)
    pltpu.sync_copy(col_h.at[pl.ds(ebase, ECHUNK)], colb)
    pltpu.sync_copy(w1_h.at[pl.ds(ebase, ECHUNK)], w1b)
    pltpu.sync_copy(w2_h.at[pl.ds(ebase, ECHUNK)], w2b)

    def wn_loop(i, _):
        sl = pl.ds(i * 16, 16)
        r = rowb[sl]
        wn1b[sl] = w1b[sl] * plsc.load_gather(dis1l, [r])
        wn2b[sl] = w2b[sl] * plsc.load_gather(dis2l, [r])
        return 0
    lax.fori_loop(0, ECHUNK // 16, wn_loop, 0)
    pltpu.sync_copy(wn2b, wn2_o.at[pl.ds(ebase, ECHUNK)])

    for f in range(5):
        foff = jnp.int32(f * NPAD)

        def msg_loop(i, _):
            sl = pl.ds(i * 16, 16)
            valb[sl] = wn1b[sl] * plsc.load_gather(xll, [rowb[sl] + foff])
            return 0
        lax.fori_loop(0, ECHUNK // 16, msg_loop, 0)
        pltpu.sync_copy(valb, msgs[f].at[colb], add=True)
    plsc.subcore_barrier()

    for f in range(5):
        moff = (c * 5 + f) * NPAD + nbase
        pltpu.sync_copy(msgs[f].at[pl.ds(nbase, NPS)],
                        msg_o.at[pl.ds(moff, NPS)])


def _sc_msg2_body(row_h, col_h, wn_h, xl2_h,
                  msg_o,
                  rowb, colb, wnb, valb, xl2l, nodeb,
                  m0s):
    c = lax.axis_index("c")
    s = lax.axis_index("s")
    wid = c * NS + s
    nbase = s * NPS

    def zero_loop(i, _):
        nodeb[pl.ds(i * 16, 16)] = jnp.zeros((16,), jnp.float32)
        return 0
    lax.fori_loop(0, NPS // 16, zero_loop, 0)
    pltpu.sync_copy(nodeb, m0s.at[pl.ds(nbase, NPS)])
    plsc.subcore_barrier()

    pltpu.sync_copy(xl2_h, xl2l)
    ebase = wid * ECHUNK
    pltpu.sync_copy(row_h.at[pl.ds(ebase, ECHUNK)], rowb)
    pltpu.sync_copy(col_h.at[pl.ds(ebase, ECHUNK)], colb)
    pltpu.sync_copy(wn_h.at[pl.ds(ebase, ECHUNK)], wnb)

    def msg_loop(i, _):
        sl = pl.ds(i * 16, 16)
        valb[sl] = wnb[sl] * plsc.load_gather(xl2l, [rowb[sl]])
        return 0
    lax.fori_loop(0, ECHUNK // 16, msg_loop, 0)
    pltpu.sync_copy(valb, m0s.at[colb], add=True)
    plsc.subcore_barrier()

    pltpu.sync_copy(m0s.at[pl.ds(nbase, NPS)],
                    msg_o.at[pl.ds(c * NPAD + nbase, NPS)])


def _sc_mesh():
    return plsc.VectorSubcoreMesh(core_axis_name="c", subcore_axis_name="s",
                                  num_cores=NC, num_subcores=NS)


_full_spec = lambda shp: pl.BlockSpec(shp, lambda: tuple(0 for _ in shp))


@jax.jit
def kernel(x, edge_index, edge_attr, batch_idx, W1, b1, mlp1_w1, mlp1_b1,
           mlp1_w2, mlp1_b2, gn_w, gn_b, gn_ms, W2, b2, mlp2_w1, mlp2_b1,
           mlp2_w2, mlp2_b2):
    f32 = jnp.float32

    row_p = jnp.pad(edge_index[0], (0, EPAD - E), constant_values=NPAD - 1)
    col_p = jnp.pad(edge_index[1], (0, EPAD - E), constant_values=NPAD - 1)
    batch_p = jnp.pad(batch_idx, (0, NPAD - N), constant_values=-1)[None, :]

    xlt = pl.pallas_call(
        _xl1_body,
        grid=(NPAD // XB,),
        in_specs=[pl.BlockSpec((XB, D), lambda i: (i, 0)),
                  pl.BlockSpec((D, 5), lambda i: (0, 0))],
        out_specs=pl.BlockSpec((5, XB), lambda i: (0, i)),
        out_shape=jax.ShapeDtypeStruct((5, NPAD), f32),
    )(x, W1)

    wcat = jnp.concatenate([mlp1_w1, mlp2_w1], axis=1)
    bcat = jnp.concatenate([mlp1_b1, mlp2_b1])[:, None]
    w2cat = jnp.zeros((10, 2), f32)
    w2cat = w2cat.at[:5, 0].set(mlp1_w2[:, 0]).at[5:, 1].set(mlp2_w2[:, 0])
    b2cat = jnp.concatenate([mlp1_b2, mlp2_b2])[:, None]

    wspec = pl.BlockSpec((1, EB), lambda i: (0, i))
    full = lambda shp: pl.BlockSpec(shp, lambda i: tuple(0 for _ in shp))
    w1e, w2e = pl.pallas_call(
        _edge_w_body,
        grid=(EPAD // EB,),
        in_specs=[pl.BlockSpec((EB, DE), lambda i: (i, 0)),
                  full((DE, 10)), full((10, 1)), full((10, 2)), full((2, 1))],
        out_specs=[wspec, wspec],
        out_shape=[jax.ShapeDtypeStruct((1, EPAD), f32),
                   jax.ShapeDtypeStruct((1, EPAD), f32)],
    )(edge_attr, wcat, bcat, w2cat, b2cat)
    w1e = w1e.reshape(EPAD)
    w2e = w2e.reshape(EPAD)

    sc_deg = pl.kernel(
        _sc_deg_body,
        out_type=[jax.ShapeDtypeStruct((NC * NPAD,), f32),
                  jax.ShapeDtypeStruct((NC * NPAD,), f32)],
        mesh=_sc_mesh(),
        compiler_params=pltpu.CompilerParams(needs_layout_passes=False),
        scratch_types=[
            pltpu.VMEM((ECHUNK,), jnp.int32),
            pltpu.VMEM((ECHUNK,), f32),
            pltpu.VMEM((ECHUNK,), f32),
            pltpu.VMEM((NPS,), f32),
            pltpu.VMEM_SHARED((NPAD,), f32),
            pltpu.VMEM_SHARED((NPAD,), f32),
        ],
    )
    deg1p, deg2p = sc_deg(col_p, w1e, w2e)

    sc1 = pl.kernel(
        _sc_msg1_body,
        out_type=[jax.ShapeDtypeStruct((NC * 5 * NPAD,), f32),
                  jax.ShapeDtypeStruct((NPAD,), f32),
                  jax.ShapeDtypeStruct((NPAD,), f32),
                  jax.ShapeDtypeStruct((EPAD,), f32)],
        mesh=_sc_mesh(),
        compiler_params=pltpu.CompilerParams(needs_layout_passes=False),
        scratch_types=[
            pltpu.VMEM((ECHUNK,), jnp.int32),
            pltpu.VMEM((ECHUNK,), jnp.int32),
            pltpu.VMEM((ECHUNK,), f32),
            pltpu.VMEM((ECHUNK,), f32),
            pltpu.VMEM((ECHUNK,), f32),
            pltpu.VMEM((ECHUNK,), f32),
            pltpu.VMEM((ECHUNK,), f32),
            pltpu.VMEM((NPAD,), f32),
            pltpu.VMEM((NPAD,), f32),
            pltpu.VMEM((5 * NPAD,), f32),
            pltpu.VMEM((NPS,), f32),
            pltpu.VMEM((NPS,), f32),
            pltpu.VMEM_SHARED((NPAD,), f32),
            pltpu.VMEM_SHARED((NPAD,), f32),
            pltpu.VMEM_SHARED((NPAD,), f32),
            pltpu.VMEM_SHARED((NPAD,), f32),
            pltpu.VMEM_SHARED((NPAD,), f32),
            pltpu.VMEM_SHARED((NPAD,), f32),
            pltpu.VMEM_SHARED((NPAD,), f32),
        ],
    )
    msg1, deg1, deg2, wn2 = sc1(row_p, col_p, w1e, w2e,
                                xlt.reshape(5 * NPAD), deg1p, deg2p)
    msg1 = msg1.reshape(NC, 5, NPAD)
    deg1 = deg1[None, :]
    deg2 = deg2[None, :]

    xl2 = pl.pallas_call(
        _mid_body,
        in_specs=[
            _full_spec((NC, 5, NPAD)),
            _full_spec((1, NPAD)),
            _full_spec((5, NPAD)),
            _full_spec((1, NPAD)),
            _full_spec((5, 1)),
            _full_spec((5, 1)),
            _full_spec((5, 1)),
            _full_spec((5, 1)),
            _full_spec((5, 1)),
        ],
        out_specs=_full_spec((1, NPAD)),
        out_shape=jax.ShapeDtypeStruct((1, NPAD), f32),
    )(msg1, deg1, xlt, batch_p, b1[:, None], gn_w[:, None],
      gn_b[:, None], gn_ms[:, None], W2)

    sc2 = pl.kernel(
        _sc_msg2_body,
        out_type=[jax.ShapeDtypeStruct((NC * NPAD,), f32)],
        mesh=_sc_mesh(),
        compiler_params=pltpu.CompilerParams(needs_layout_passes=False),
        scratch_types=[
            pltpu.VMEM((ECHUNK,), jnp.int32),
            pltpu.VMEM((ECHUNK,), jnp.int32),
            pltpu.VMEM((ECHUNK,), f32),
            pltpu.VMEM((ECHUNK,), f32),
            pltpu.VMEM((NPAD,), f32),
            pltpu.VMEM((NPS,), f32),
            pltpu.VMEM_SHARED((NPAD,), f32),
        ],
    )
    (msg2,) = sc2(row_p, col_p, wn2, xl2.reshape(NPAD))
    msg2 = msg2.reshape(NC, NPAD)

    out = pl.pallas_call(
        _final_body,
        in_specs=[_full_spec((NC, NPAD)), _full_spec((1, NPAD)),
                  _full_spec((1, NPAD)), _full_spec((1, 1))],
        out_specs=_full_spec((1, NPAD)),
        out_shape=jax.ShapeDtypeStruct((1, NPAD), f32),
    )(msg2, deg2, xl2, b2[:, None])

    return out[0, :N, None]
